# baseline (speedup 1.0000x reference)
import functools

import jax
import jax.numpy as jnp
from jax import lax
from jax.experimental import pallas as pl
from jax.experimental.pallas import tpu as pltpu
from jax.experimental.pallas import tpu_sc as plsc

N = 10000
E = 320000
IN_DIM = 128
HID = 64
OUT_DIM = 128
K = 500
KP = 512
ABINS = KP * KP

NC = 2
NS = 16
NW = NC * NS
L = 16

EC = 80
EPW = E // NW
ECH = EPW // EC
NCH = N // EC
U = 5

_MESH = plsc.VectorSubcoreMesh(
    core_axis_name="c", subcore_axis_name="s", num_cores=NC, num_subcores=NS)
_SC_PARAMS = pltpu.CompilerParams(
    needs_layout_passes=False, use_tc_tiling_on_sc=False)

_HIGHEST = lax.Precision.HIGHEST


def _worker_ids():
    cidx = lax.axis_index("c")
    sid = lax.axis_index("s")
    return cidx, sid, cidx * NS + sid


def _fill(ref, n, value):
    for t in range(n // L):
        ref[pl.ds(t * L, L)] = jnp.full((L,), value, dtype=ref.dtype)


def _zero2d(ref):
    for r in range(ref.shape[0]):
        for t in range(ref.shape[1] // L):
            ref[r, pl.ds(t * L, L)] = jnp.zeros((L,), jnp.float32)


@functools.partial(
    pl.kernel,
    out_type=[
        jax.ShapeDtypeStruct((NC, ABINS), jnp.float32),
        jax.ShapeDtypeStruct((NC, 10240), jnp.float32),
    ],
    mesh=_MESH,
    compiler_params=_SC_PARAMS,
    scratch_types=[
        pltpu.VMEM((N,), jnp.int32),
        pltpu.VMEM((ECH, EC), jnp.int32),
        pltpu.VMEM((ECH, EC), jnp.int32),
        pltpu.VMEM((U, EC), jnp.int32),
        pltpu.VMEM((EC,), jnp.float32),
        pltpu.VMEM((1024,), jnp.float32),
        pltpu.VMEM_SHARED((ABINS,), jnp.float32),
        pltpu.VMEM_SHARED((10240,), jnp.float32),
    ] + [pltpu.SemaphoreType.DMA] * (2 * U),
)
def _sc_stats(src_hbm, dst_hbm, cid_hbm, a_out, deg_out,
              cid_v, srcs_v, dsts_v, code_v, ones_v, zb_v,
              a_sh, deg_sh, *sems):
    sa = sems[:U]
    sd = sems[U:2 * U]
    cidx, sid, gw = _worker_ids()
    _fill(zb_v, 1024, 0.0)
    _fill(ones_v, EC, 1.0)

    def _zero_a(i, carry):
        pltpu.sync_copy(zb_v, a_sh.at[pl.ds(i * 1024, 1024)])
        return carry
    lax.fori_loop(0, ABINS // (1024 * NS), lambda i, c: _zero_a(i * NS + sid, c), 0)

    @pl.when(sid < 10)
    def _():
        pltpu.sync_copy(zb_v, deg_sh.at[pl.ds(sid * 1024, 1024)])

    pltpu.sync_copy(cid_hbm, cid_v)
    pltpu.sync_copy(src_hbm.at[gw], srcs_v)
    pltpu.sync_copy(dst_hbm.at[gw], dsts_v)
    plsc.subcore_barrier()

    def _body(m, carry):
        descs = []
        for s in range(U):
            j = m * U + s
            for t in range(EC // L):
                s16 = srcs_v[j, pl.ds(t * L, L)]
                d16 = dsts_v[j, pl.ds(t * L, L)]
                cu = plsc.load_gather(cid_v, [s16])
                cv = plsc.load_gather(cid_v, [d16])
                code_v[s, pl.ds(t * L, L)] = cu * KP + cv
            descs.append(pltpu.async_copy(
                ones_v, a_sh.at[code_v.at[s]], sa[s], add=True))
            descs.append(pltpu.async_copy(
                ones_v, deg_sh.at[dsts_v.at[j]], sd[s], add=True))
        for d in descs:
            d.wait()
        return carry
    lax.fori_loop(0, ECH // U, _body, 0)

    plsc.subcore_barrier()
    chunk = ABINS // NS
    pltpu.sync_copy(a_sh.at[pl.ds(sid * chunk, chunk)],
                    a_out.at[cidx, pl.ds(sid * chunk, chunk)])
    pltpu.sync_copy(deg_sh.at[pl.ds(sid * 640, 640)],
                    deg_out.at[cidx, pl.ds(sid * 640, 640)])


@functools.partial(
    pl.kernel,
    out_type=jax.ShapeDtypeStruct((NC, N, HID), jnp.float32),
    mesh=_MESH,
    compiler_params=_SC_PARAMS,
    scratch_types=[
        pltpu.VMEM((ECH, EC), jnp.int32),
        pltpu.VMEM((ECH, EC), jnp.int32),
        pltpu.VMEM((U, EC, HID), jnp.float32),
        pltpu.VMEM((L, HID), jnp.float32),
        pltpu.VMEM_SHARED((N, HID), jnp.float32),
    ] + [pltpu.SemaphoreType.DMA] * (2 * U),
)
def _sc_message(src_hbm, dst_hbm, y_hbm, acc_out,
                srcs_v, dsts_v, rows_v, zb_v, acc_sh, *sems):
    sg = sems[:U]
    ss = sems[U:2 * U]
    cidx, sid, gw = _worker_ids()
    _zero2d(zb_v)

    def _zero(i, carry):
        pltpu.sync_copy(zb_v, acc_sh.at[pl.ds(i * L, L), :])
        return carry
    lax.fori_loop(0, (N // L) // NS, lambda i, c: _zero(i * NS + sid, c), 0)
    pltpu.sync_copy(src_hbm.at[gw], srcs_v)
    pltpu.sync_copy(dst_hbm.at[gw], dsts_v)
    plsc.subcore_barrier()

    def _body(m, carry):
        gd = [pltpu.async_copy(y_hbm.at[srcs_v.at[m * U + s]],
                               rows_v.at[s], sg[s])
              for s in range(U)]
        sd = []
        for s in range(U):
            gd[s].wait()
            sd.append(pltpu.async_copy(
                rows_v.at[s], acc_sh.at[dsts_v.at[m * U + s]], ss[s], add=True))
        for d in sd:
            d.wait()
        return carry
    lax.fori_loop(0, ECH // U, _body, 0)

    plsc.subcore_barrier()

    def _flush(i, carry):
        pltpu.sync_copy(acc_sh.at[pl.ds(i * 200, 200), :],
                        acc_out.at[cidx, pl.ds(i * 200, 200), :])
        return carry
    lax.fori_loop(0, (N // 200 - sid + NS - 1) // NS,
                  lambda i, c: _flush(sid + i * NS, c), 0)


def _tc1_body(x_ref, w1_ref, deg_ref, y_ref, dinv_ref):
    deg = deg_ref[0, :N] + deg_ref[1, :N] + 1.0
    dinv = lax.rsqrt(deg)[:, None]
    xw = jnp.dot(x_ref[...], w1_ref[...], precision=_HIGHEST)
    y_ref[...] = dinv * xw
    dinv_ref[...] = dinv


RB = 2000


def _tc2_body(acc_ref, y_ref, dinv_ref, cid_ref, b1_ref, wg_ref, ws_ref,
              bs_ref, sums_ref, cnt_ref, skip_ref):
    i = pl.program_id(0)
    dinv = dinv_ref[...]
    msg = acc_ref[0] + acc_ref[1] + y_ref[...]
    x1 = jnp.maximum(dinv * msg + b1_ref[...][None, :], 0.0)
    gate = jnp.tanh(jnp.dot(x1, wg_ref[...], precision=_HIGHEST))
    x1g = x1 * gate
    kk = lax.broadcasted_iota(jnp.int32, (RB, KP), 1)
    onehot = jnp.where(cid_ref[...] == kk, 1.0, 0.0)
    psums = lax.dot_general(
        onehot, x1g, (((0,), (0,)), ((), ())), precision=_HIGHEST)
    pcnt = jnp.sum(onehot, axis=0)[:, None]

    @pl.when(i == 0)
    def _():
        sums_ref[...] = jnp.zeros_like(sums_ref)
        cnt_ref[...] = jnp.zeros_like(cnt_ref)
    sums_ref[...] += psums
    cnt_ref[...] += pcnt
    skip_ref[...] = jnp.dot(x1, ws_ref[...], precision=_HIGHEST) + bs_ref[...][None, :]


def _tc3_body(a_ref, sums_ref, cnt_ref, cid_ref, skip_ref, w2_ref, b2_ref,
              out_ref):
    a = a_ref[0] + a_ref[1]
    ii = lax.broadcasted_iota(jnp.int32, (KP, KP), 0)
    jj = lax.broadcasted_iota(jnp.int32, (KP, KP), 1)
    a = jnp.where(ii == jj, 0.0, a)
    degp = jnp.sum(a, axis=0) + 1.0
    dinvp = lax.rsqrt(degp)[:, None]
    xpool = sums_ref[...] / jnp.maximum(cnt_ref[...], 1.0)
    xw2 = jnp.dot(xpool, w2_ref[...], precision=_HIGHEST)
    y2 = dinvp * xw2
    t = lax.dot_general(a, y2, (((0,), (0,)), ((), ())), precision=_HIGHEST)
    xp = dinvp * (t + y2) + b2_ref[...][None, :]
    kk = lax.broadcasted_iota(jnp.int32, (RB, KP), 1)
    onehot = jnp.where(cid_ref[...] == kk, 1.0, 0.0)
    up = jnp.dot(onehot, xp, precision=_HIGHEST)
    out_ref[...] = up + skip_ref[...]


_tc1 = pl.pallas_call(
    _tc1_body,
    out_shape=[jax.ShapeDtypeStruct((N, HID), jnp.float32),
               jax.ShapeDtypeStruct((N, 1), jnp.float32)])

_tc2 = pl.pallas_call(
    _tc2_body,
    grid=(N // RB,),
    in_specs=[
        pl.BlockSpec((NC, RB, HID), lambda i: (0, i, 0)),
        pl.BlockSpec((RB, HID), lambda i: (i, 0)),
        pl.BlockSpec((RB, 1), lambda i: (i, 0)),
        pl.BlockSpec((RB, 1), lambda i: (i, 0)),
        pl.BlockSpec((HID,), lambda i: (0,)),
        pl.BlockSpec((HID, 1), lambda i: (0, 0)),
        pl.BlockSpec((HID, OUT_DIM), lambda i: (0, 0)),
        pl.BlockSpec((OUT_DIM,), lambda i: (0,)),
    ],
    out_specs=[
        pl.BlockSpec((KP, HID), lambda i: (0, 0)),
        pl.BlockSpec((KP, 1), lambda i: (0, 0)),
        pl.BlockSpec((RB, OUT_DIM), lambda i: (i, 0)),
    ],
    out_shape=[jax.ShapeDtypeStruct((KP, HID), jnp.float32),
               jax.ShapeDtypeStruct((KP, 1), jnp.float32),
               jax.ShapeDtypeStruct((N, OUT_DIM), jnp.float32)])

_tc3 = pl.pallas_call(
    _tc3_body,
    grid=(N // RB,),
    in_specs=[
        pl.BlockSpec((NC, KP, KP), lambda i: (0, 0, 0)),
        pl.BlockSpec((KP, HID), lambda i: (0, 0)),
        pl.BlockSpec((KP, 1), lambda i: (0, 0)),
        pl.BlockSpec((RB, 1), lambda i: (i, 0)),
        pl.BlockSpec((RB, OUT_DIM), lambda i: (i, 0)),
        pl.BlockSpec((HID, OUT_DIM), lambda i: (0, 0)),
        pl.BlockSpec((OUT_DIM,), lambda i: (0,)),
    ],
    out_specs=pl.BlockSpec((RB, OUT_DIM), lambda i: (i, 0)),
    out_shape=jax.ShapeDtypeStruct((N, OUT_DIM), jnp.float32))


def kernel(x, edge_index, cluster_id, W1, b1, W2, b2, Ws, bs, Wg):
    e3 = edge_index.reshape(2, NW, ECH, EC)
    src3 = e3[0]
    dst3 = e3[1]
    cid2 = cluster_id[:, None]
    a_part, deg_part = _sc_stats(src3, dst3, cluster_id)
    y, dinv = _tc1(x, W1, deg_part)
    acc_part = _sc_message(src3, dst3, y)
    sums, cnt, skip = _tc2(acc_part, y, dinv, cid2, b1, Wg, Ws, bs)
    a_part = a_part.reshape(NC, KP, KP)
    logits = _tc3(a_part, sums, cnt, cid2, skip, W2, b2)
    return (logits, 0.0)


# speedup vs baseline: 1.1250x; 1.1250x over previous
import functools

import jax
import jax.numpy as jnp
from jax import lax
from jax.experimental import pallas as pl
from jax.experimental.pallas import tpu as pltpu
from jax.experimental.pallas import tpu_sc as plsc

N = 10000
E = 320000
IN_DIM = 128
HID = 64
OUT_DIM = 128
K = 500
KP = 512
ABINS = KP * KP

NC = 2
NS = 16
NW = NC * NS
L = 16

EC = 80
EPW = E // NW
ECH = EPW // EC
NCH = N // EC
U = 5

_MESH = plsc.VectorSubcoreMesh(
    core_axis_name="c", subcore_axis_name="s", num_cores=NC, num_subcores=NS)
_SC_PARAMS = pltpu.CompilerParams(
    needs_layout_passes=False, use_tc_tiling_on_sc=False)

_HIGHEST = lax.Precision.HIGHEST


def _worker_ids():
    cidx = lax.axis_index("c")
    sid = lax.axis_index("s")
    return cidx, sid, cidx * NS + sid


def _fill(ref, n, value):
    for t in range(n // L):
        ref[pl.ds(t * L, L)] = jnp.full((L,), value, dtype=ref.dtype)


def _zero2d(ref):
    for r in range(ref.shape[0]):
        for t in range(ref.shape[1] // L):
            ref[r, pl.ds(t * L, L)] = jnp.zeros((L,), jnp.float32)


@functools.partial(
    pl.kernel,
    out_type=[
        jax.ShapeDtypeStruct((NC, ABINS), jnp.float32),
        jax.ShapeDtypeStruct((NC, 10240), jnp.float32),
        jax.ShapeDtypeStruct((NC, 1024), jnp.float32),
    ],
    mesh=_MESH,
    compiler_params=_SC_PARAMS,
    scratch_types=[
        pltpu.VMEM((N,), jnp.int32),
        pltpu.VMEM((ECH, EC), jnp.int32),
        pltpu.VMEM((ECH, EC), jnp.int32),
        pltpu.VMEM((U, EC), jnp.int32),
        pltpu.VMEM((EC,), jnp.int32),
        pltpu.VMEM((EC,), jnp.float32),
        pltpu.VMEM((1024,), jnp.float32),
        pltpu.VMEM_SHARED((ABINS,), jnp.float32),
        pltpu.VMEM_SHARED((10240,), jnp.float32),
        pltpu.VMEM_SHARED((1024,), jnp.float32),
    ] + [pltpu.SemaphoreType.DMA] * (2 * U),
)
def _sc_stats(src_hbm, dst_hbm, cid_hbm, a_out, deg_out, cnt_out,
              cid_v, srcs_v, dsts_v, code_v, nidx_v, ones_v, zb_v,
              a_sh, deg_sh, cnt_sh, *sems):
    sa = sems[:U]
    sd = sems[U:2 * U]
    cidx, sid, gw = _worker_ids()
    _fill(zb_v, 1024, 0.0)
    _fill(ones_v, EC, 1.0)

    def _zero_a(i, carry):
        pltpu.sync_copy(zb_v, a_sh.at[pl.ds(i * 1024, 1024)])
        return carry
    lax.fori_loop(0, ABINS // (1024 * NS), lambda i, c: _zero_a(i * NS + sid, c), 0)

    @pl.when(sid < 10)
    def _():
        pltpu.sync_copy(zb_v, deg_sh.at[pl.ds(sid * 1024, 1024)])

    @pl.when(sid == 15)
    def _():
        pltpu.sync_copy(zb_v, cnt_sh)

    pltpu.sync_copy(cid_hbm, cid_v)
    pltpu.sync_copy(src_hbm.at[gw], srcs_v)
    pltpu.sync_copy(dst_hbm.at[gw], dsts_v)
    plsc.subcore_barrier()

    def _body(m, carry):
        descs = []
        for s in range(U):
            j = m * U + s
            for t in range(EC // L):
                s16 = srcs_v[j, pl.ds(t * L, L)]
                d16 = dsts_v[j, pl.ds(t * L, L)]
                cu = plsc.load_gather(cid_v, [s16])
                cv = plsc.load_gather(cid_v, [d16])
                code_v[s, pl.ds(t * L, L)] = cu * KP + cv
            descs.append(pltpu.async_copy(
                ones_v, a_sh.at[code_v.at[s]], sa[s], add=True))
            descs.append(pltpu.async_copy(
                ones_v, deg_sh.at[dsts_v.at[j]], sd[s], add=True))
        for d in descs:
            d.wait()
        return carry
    lax.fori_loop(0, ECH // U, _body, 0)

    def _nodes(i, carry):
        pltpu.sync_copy(cid_hbm.at[pl.ds(i * EC, EC)], nidx_v)
        pltpu.sync_copy(ones_v, cnt_sh.at[nidx_v], add=True)
        return carry
    lax.fori_loop(0, (NCH - gw + NW - 1) // NW,
                  lambda i, c: _nodes(gw + i * NW, c), 0)

    plsc.subcore_barrier()
    chunk = ABINS // NS
    pltpu.sync_copy(a_sh.at[pl.ds(sid * chunk, chunk)],
                    a_out.at[cidx, pl.ds(sid * chunk, chunk)])
    pltpu.sync_copy(deg_sh.at[pl.ds(sid * 640, 640)],
                    deg_out.at[cidx, pl.ds(sid * 640, 640)])

    @pl.when(sid == 0)
    def _():
        pltpu.sync_copy(cnt_sh, cnt_out.at[cidx])


@functools.partial(
    pl.kernel,
    out_type=jax.ShapeDtypeStruct((NC, N, HID), jnp.float32),
    mesh=_MESH,
    compiler_params=_SC_PARAMS,
    scratch_types=[
        pltpu.VMEM((ECH, EC), jnp.int32),
        pltpu.VMEM((ECH, EC), jnp.int32),
        pltpu.VMEM((2 * U, EC, HID), jnp.float32),
        pltpu.VMEM((L, HID), jnp.float32),
        pltpu.VMEM_SHARED((N, HID), jnp.float32),
    ] + [pltpu.SemaphoreType.DMA] * (4 * U),
)
def _sc_message(src_hbm, dst_hbm, y_hbm, acc_out,
                srcs_v, dsts_v, rows_v, zb_v, acc_sh, *sems):
    sg = sems[:2 * U]
    ss = sems[2 * U:4 * U]
    cidx, sid, gw = _worker_ids()
    _zero2d(zb_v)

    def _zero(i, carry):
        pltpu.sync_copy(zb_v, acc_sh.at[pl.ds(i * L, L), :])
        return carry
    lax.fori_loop(0, (N // L) // NS, lambda i, c: _zero(i * NS + sid, c), 0)
    pltpu.sync_copy(src_hbm.at[gw], srcs_v)
    pltpu.sync_copy(dst_hbm.at[gw], dsts_v)
    plsc.subcore_barrier()

    def _gathers(j0, p):
        return [pltpu.async_copy(y_hbm.at[srcs_v.at[j0 + s]],
                                 rows_v.at[p * U + s], sg[p * U + s])
                for s in range(U)]

    def _scatters(j0, p, gd):
        out = []
        for s in range(U):
            gd[s].wait()
            out.append(pltpu.async_copy(
                rows_v.at[p * U + s], acc_sh.at[dsts_v.at[j0 + s]],
                ss[p * U + s], add=True))
        return out

    def _body(m2, carry):
        ja = 2 * m2 * U
        jb = ja + U
        ga = _gathers(ja, 0)
        sa_d = _scatters(ja, 0, ga)
        gb = _gathers(jb, 1)
        for d in sa_d:
            d.wait()
        sb_d = _scatters(jb, 1, gb)
        for d in sb_d:
            d.wait()
        return carry
    lax.fori_loop(0, ECH // (2 * U), _body, 0)

    jt = (ECH // (2 * U)) * 2 * U
    gt = _gathers(jt, 0)
    st = _scatters(jt, 0, gt)
    for d in st:
        d.wait()

    plsc.subcore_barrier()

    def _flush(i, carry):
        pltpu.sync_copy(acc_sh.at[pl.ds(i * 200, 200), :],
                        acc_out.at[cidx, pl.ds(i * 200, 200), :])
        return carry
    lax.fori_loop(0, (N // 200 - sid + NS - 1) // NS,
                  lambda i, c: _flush(sid + i * NS, c), 0)


@functools.partial(
    pl.kernel,
    out_type=jax.ShapeDtypeStruct((NC, KP, HID), jnp.float32),
    mesh=_MESH,
    compiler_params=_SC_PARAMS,
    scratch_types=[
        pltpu.VMEM((EC,), jnp.int32),
        pltpu.VMEM((EC, HID), jnp.float32),
        pltpu.VMEM((L, HID), jnp.float32),
        pltpu.VMEM_SHARED((KP, HID), jnp.float32),
    ],
)
def _sc_pool(cid_hbm, x1g_hbm, sums_out, idx_v, rows_v, zb_v, sums_sh):
    cidx, sid, gw = _worker_ids()
    _zero2d(zb_v)

    def _zero(i, carry):
        pltpu.sync_copy(zb_v, sums_sh.at[pl.ds(i * L, L), :])
        return carry
    lax.fori_loop(0, (KP // L) // NS, lambda i, c: _zero(i * NS + sid, c), 0)
    plsc.subcore_barrier()

    def _nodes(i, carry):
        base = i * EC
        pltpu.sync_copy(cid_hbm.at[pl.ds(base, EC)], idx_v)
        pltpu.sync_copy(x1g_hbm.at[pl.ds(base, EC), :], rows_v)
        pltpu.sync_copy(rows_v, sums_sh.at[idx_v], add=True)
        return carry
    lax.fori_loop(0, (NCH - gw + NW - 1) // NW,
                  lambda i, c: _nodes(gw + i * NW, c), 0)

    plsc.subcore_barrier()
    rows = KP // NS
    pltpu.sync_copy(sums_sh.at[pl.ds(sid * rows, rows), :],
                    sums_out.at[cidx, pl.ds(sid * rows, rows), :])


@functools.partial(
    pl.kernel,
    out_type=jax.ShapeDtypeStruct((N, OUT_DIM), jnp.float32),
    mesh=_MESH,
    compiler_params=_SC_PARAMS,
    scratch_types=[
        pltpu.VMEM((EC,), jnp.int32),
        pltpu.VMEM((EC, OUT_DIM), jnp.float32),
        pltpu.VMEM((EC, OUT_DIM), jnp.float32),
        pltpu.SemaphoreType.DMA,
        pltpu.SemaphoreType.DMA,
        pltpu.SemaphoreType.DMA,
    ],
)
def _sc_gather(cid_hbm, xp_hbm, skip_hbm, out_hbm, idx_v, rows_v, skip_v,
               sem_i, sem_g, sem_s):
    cidx, sid, gw = _worker_ids()

    def _nodes(i, carry):
        base = i * EC
        di = pltpu.async_copy(cid_hbm.at[pl.ds(base, EC)], idx_v, sem_i)
        dsk = pltpu.async_copy(skip_hbm.at[pl.ds(base, EC), :], skip_v, sem_s)
        di.wait()
        dg = pltpu.async_copy(xp_hbm.at[idx_v], rows_v, sem_g)
        dsk.wait()
        dg.wait()

        def _row(r, c2):
            for t in range(OUT_DIM // L):
                sl = pl.ds(t * L, L)
                rows_v[r, sl] = rows_v[r, sl] + skip_v[r, sl]
            return c2
        lax.fori_loop(0, EC, _row, 0)
        pltpu.sync_copy(rows_v, out_hbm.at[pl.ds(base, EC), :])
        return carry
    lax.fori_loop(0, (NCH - gw + NW - 1) // NW,
                  lambda i, c: _nodes(gw + i * NW, c), 0)


def _tc1_body(x_ref, w1_ref, deg_ref, y_ref, dinv_ref):
    deg = deg_ref[0, :N] + deg_ref[1, :N] + 1.0
    dinv = lax.rsqrt(deg)[:, None]
    xw = jnp.dot(x_ref[...], w1_ref[...], precision=_HIGHEST)
    y_ref[...] = dinv * xw
    dinv_ref[...] = dinv


def _tc2_body(acc_ref, y_ref, dinv_ref, b1_ref, wg_ref, ws_ref, bs_ref,
              x1g_ref, skip_ref):
    dinv = dinv_ref[...]
    msg = acc_ref[0] + acc_ref[1] + y_ref[...]
    x1 = jnp.maximum(dinv * msg + b1_ref[...][None, :], 0.0)
    gate = jnp.tanh(jnp.dot(x1, wg_ref[...], precision=_HIGHEST))
    x1g_ref[...] = x1 * gate
    skip_ref[...] = jnp.dot(x1, ws_ref[...], precision=_HIGHEST) + bs_ref[...][None, :]


def _tc3_body(a_ref, sums_ref, cnt_ref, w2_ref, b2_ref, xp_ref):
    a = a_ref[0] + a_ref[1]
    ii = lax.broadcasted_iota(jnp.int32, (KP, KP), 0)
    jj = lax.broadcasted_iota(jnp.int32, (KP, KP), 1)
    a = jnp.where(ii == jj, 0.0, a)
    degp = jnp.sum(a, axis=0) + 1.0
    dinvp = lax.rsqrt(degp)[:, None]
    cnt = cnt_ref[0, :KP] + cnt_ref[1, :KP]
    xpool = ((sums_ref[0] + sums_ref[1])
             / jnp.maximum(cnt, 1.0)[:, None])
    xw2 = jnp.dot(xpool, w2_ref[...], precision=_HIGHEST)
    y2 = dinvp * xw2
    t = lax.dot_general(a, y2, (((0,), (0,)), ((), ())), precision=_HIGHEST)
    xp_ref[...] = dinvp * (t + y2) + b2_ref[...][None, :]


_tc1 = pl.pallas_call(
    _tc1_body,
    out_shape=[jax.ShapeDtypeStruct((N, HID), jnp.float32),
               jax.ShapeDtypeStruct((N, 1), jnp.float32)])

_tc2 = pl.pallas_call(
    _tc2_body,
    out_shape=[jax.ShapeDtypeStruct((N, HID), jnp.float32),
               jax.ShapeDtypeStruct((N, OUT_DIM), jnp.float32)])

_tc3 = pl.pallas_call(
    _tc3_body,
    out_shape=jax.ShapeDtypeStruct((KP, OUT_DIM), jnp.float32))


def kernel(x, edge_index, cluster_id, W1, b1, W2, b2, Ws, bs, Wg):
    e3 = edge_index.reshape(2, NW, ECH, EC)
    src3 = e3[0]
    dst3 = e3[1]
    a_part, deg_part, cnt_part = _sc_stats(src3, dst3, cluster_id)
    y, dinv = _tc1(x, W1, deg_part)
    acc_part = _sc_message(src3, dst3, y)
    x1g, skip = _tc2(acc_part, y, dinv, b1, Wg, Ws, bs)
    sums_part = _sc_pool(cluster_id, x1g)
    a_part = a_part.reshape(NC, KP, KP)
    xp = _tc3(a_part, sums_part, cnt_part, W2, b2)
    logits = _sc_gather(cluster_id, xp, skip)
    return (logits, 0.0)
